# Initial kernel scaffold; baseline (speedup 1.0000x reference)
import jax
import jax.numpy as jnp
from jax.experimental import pallas as pl


def kernel(hiddens, cells, hidden_masks, op, init_hidden, init_cell):
    raise NotImplementedError("write your pallas kernel here")


# speedup vs baseline: 2.4422x; 2.4422x over previous
import jax
import jax.numpy as jnp
from jax import lax
from jax.experimental import pallas as pl
from jax.experimental.pallas import tpu as pltpu
from jax.experimental.pallas import tpu_sc as plsc

SEQ = 2048
B = 16
H = 512


def _column_sum(masks_v):
    v = [jnp.zeros((B,), jnp.float32) for _ in range(8)]
    for j in range(16):
        def body(i, accs):
            base = 128 * i + 8 * j
            return tuple(
                accs[s] + masks_v[pl.ds((base + s) * B, B)] for s in range(8)
            )
        accs = lax.fori_loop(
            0, 16, body,
            tuple(jnp.zeros((B,), jnp.float32) for _ in range(8)),
        )
        v = [v[s] + accs[s] for s in range(8)]
    b4 = [v[s] + v[s + 4] for s in range(4)]
    c2 = [b4[s] + b4[s + 2] for s in range(2)]
    return c2[0] + c2[1]


def _make_kernel():
    mesh = plsc.VectorSubcoreMesh(core_axis_name="c", subcore_axis_name="s")

    def body(h_hbm, c_hbm, masks_hbm, op_hbm, ih_hbm, ic_hbm,
             out_h, out_c, masks_v, op_v, init_v, idx_v, rows_v, sem):
        wid = lax.axis_index("s") * 2 + lax.axis_index("c")

        def work(table, init_ref, out_ref):
            pltpu.sync_copy(masks_hbm, masks_v)
            pltpu.sync_copy(op_hbm, op_v)
            pltpu.sync_copy(init_ref, init_v)
            pos = _column_sum(masks_v).astype(jnp.int32)
            opv = op_v[...]
            prev = jnp.where(pos == 0, SEQ, pos - 1)
            sel = jnp.where(opv != 0, prev, pos)
            lane = lax.iota(jnp.int32, 16)
            idx_v[...] = jnp.maximum(sel - 1, 0) * B + lane
            pltpu.async_copy(table.at[idx_v], rows_v, sem).wait()
            for b in range(B):
                @pl.when(sel[b] == 0)
                def _():
                    pltpu.sync_copy(init_ref, rows_v.at[b])
            pltpu.sync_copy(rows_v, out_ref)

        @pl.when(wid == 0)
        def _():
            work(h_hbm, ih_hbm, out_h)

        @pl.when(wid == 1)
        def _():
            work(c_hbm, ic_hbm, out_c)

    return pl.kernel(
        body,
        mesh=mesh,
        out_type=(
            jax.ShapeDtypeStruct((B, H), jnp.float32),
            jax.ShapeDtypeStruct((B, H), jnp.float32),
        ),
        scratch_types=[
            pltpu.VMEM((SEQ * B,), jnp.float32),
            pltpu.VMEM((B,), jnp.int32),
            pltpu.VMEM((H,), jnp.float32),
            pltpu.VMEM((B,), jnp.int32),
            pltpu.VMEM((B, H), jnp.float32),
            pltpu.SemaphoreType.DMA,
        ],
    )


_sc_kernel = _make_kernel()


def kernel(hiddens, cells, hidden_masks, op, init_hidden, init_cell):
    h2 = hiddens.reshape(SEQ * B, H)
    c2 = cells.reshape(SEQ * B, H)
    m1 = hidden_masks.reshape(SEQ * B)
    return _sc_kernel(h2, c2, m1, op, init_hidden, init_cell)


# speedup vs baseline: 2.9846x; 1.2221x over previous
import jax
import jax.numpy as jnp
from jax import lax
from jax.experimental import pallas as pl
from jax.experimental.pallas import tpu as pltpu
from jax.experimental.pallas import tpu_sc as plsc

SEQ = 2048
B = 16
H = 512
NG = 16
NS = 8


def _make_kernel():
    mesh = plsc.VectorSubcoreMesh(core_axis_name="c", subcore_axis_name="s")

    def body(h_hbm, c_hbm, masks_hbm, op_hbm, ih_hbm, ic_hbm,
             out_h, out_c,
             stripe_v, acc_v, comb_v, op_v, init_v, idx_v, rows_v,
             shared, sem):
        cid = lax.axis_index("c")
        sid = lax.axis_index("s")

        copies = [
            pltpu.async_copy(
                masks_hbm.at[pl.ds((128 * i + NS * sid) * B, NS * B)],
                stripe_v.at[pl.ds(i * NS * B, NS * B)],
                sem,
            )
            for i in range(16)
        ]
        for cp in copies:
            cp.wait()

        def acc_body(i, accs):
            return tuple(
                accs[s] + stripe_v[pl.ds((i * NS + s) * B, B)]
                for s in range(NS)
            )
        accs = lax.fori_loop(
            0, 16, acc_body,
            tuple(jnp.zeros((B,), jnp.float32) for _ in range(NS)),
        )
        for s in range(NS):
            acc_v[pl.ds(s * B, B)] = accs[s]
        pltpu.sync_copy(acc_v, shared.at[sid])
        plsc.subcore_barrier()

        @pl.when(sid == 0)
        def _():
            pltpu.sync_copy(shared, comb_v)
            pltpu.sync_copy(op_hbm, op_v)

            def comb_body(j, vs):
                return tuple(
                    vs[s] + comb_v[j, pl.ds(s * B, B)] for s in range(NS)
                )
            v = lax.fori_loop(
                0, NG, comb_body,
                tuple(jnp.zeros((B,), jnp.float32) for _ in range(NS)),
            )
            b4 = [v[s] + v[s + 4] for s in range(4)]
            c2 = [b4[s] + b4[s + 2] for s in range(2)]
            pos = (c2[0] + c2[1]).astype(jnp.int32)

            opv = op_v[...]
            prev = jnp.where(pos == 0, SEQ, pos - 1)
            sel = jnp.where(opv != 0, prev, pos)
            lane = lax.iota(jnp.int32, 16)

            def finish(table, init_ref, out_ref):
                idx_v[...] = jnp.maximum(sel - 1, 0) * B + lane
                pltpu.async_copy(table.at[idx_v], rows_v, sem).wait()

                for b in range(B):
                    @pl.when(sel[b] == 0)
                    def _():
                        pltpu.sync_copy(init_ref, rows_v.at[b])

                pltpu.sync_copy(rows_v, out_ref)

            @pl.when(cid == 0)
            def _():
                finish(h_hbm, ih_hbm, out_h)

            @pl.when(cid == 1)
            def _():
                finish(c_hbm, ic_hbm, out_c)

    return pl.kernel(
        body,
        mesh=mesh,
        out_type=(
            jax.ShapeDtypeStruct((B, H), jnp.float32),
            jax.ShapeDtypeStruct((B, H), jnp.float32),
        ),
        scratch_types=[
            pltpu.VMEM((16 * NS * B,), jnp.float32),
            pltpu.VMEM((NS * B,), jnp.float32),
            pltpu.VMEM((NG, NS * B), jnp.float32),
            pltpu.VMEM((B,), jnp.int32),
            pltpu.VMEM((H,), jnp.float32),
            pltpu.VMEM((B,), jnp.int32),
            pltpu.VMEM((B, H), jnp.float32),
            pltpu.MemorySpace.VMEM_SHARED((NG, NS * B), jnp.float32),
            pltpu.SemaphoreType.DMA,
        ],
    )


_sc_kernel = _make_kernel()


def kernel(hiddens, cells, hidden_masks, op, init_hidden, init_cell):
    h2 = hiddens.reshape(SEQ * B, H)
    c2 = cells.reshape(SEQ * B, H)
    m1 = hidden_masks.reshape(SEQ * B)
    return _sc_kernel(h2, c2, m1, op, init_hidden, init_cell)


# speedup vs baseline: 3.1098x; 1.0419x over previous
import jax
import jax.numpy as jnp
from jax import lax
from jax.experimental import pallas as pl
from jax.experimental.pallas import tpu as pltpu
from jax.experimental.pallas import tpu_sc as plsc

SEQ = 2048
B = 16
H = 512
NG = 16
NS = 8


def _make_kernel():
    mesh = plsc.VectorSubcoreMesh(core_axis_name="c", subcore_axis_name="s")

    def body(h_hbm, c_hbm, masks_hbm, op_hbm, ih_hbm, ic_hbm,
             out_h, out_c,
             stripe_v, midx_v, acc_v, comb_v, op_v, idx_v, rows_v,
             shared, sem):
        cid = lax.axis_index("c")
        sid = lax.axis_index("s")
        lane = lax.iota(jnp.int32, 16)

        midx_v[...] = lane * NG + sid
        pltpu.async_copy(masks_hbm.at[midx_v], stripe_v, sem).wait()

        def acc_body(i, accs):
            return tuple(
                accs[s] + stripe_v[i, pl.ds(s * B, B)] for s in range(NS)
            )
        accs = lax.fori_loop(
            0, 16, acc_body,
            tuple(jnp.zeros((B,), jnp.float32) for _ in range(NS)),
        )
        for s in range(NS):
            acc_v[pl.ds(s * B, B)] = accs[s]
        pltpu.sync_copy(acc_v, shared.at[sid])
        plsc.subcore_barrier()

        @pl.when(sid == 0)
        def _():
            pltpu.sync_copy(shared, comb_v)
            pltpu.sync_copy(op_hbm, op_v)

            def comb_body(j, vs):
                return tuple(
                    vs[s] + comb_v[j, pl.ds(s * B, B)] for s in range(NS)
                )
            v = lax.fori_loop(
                0, NG, comb_body,
                tuple(jnp.zeros((B,), jnp.float32) for _ in range(NS)),
            )
            b4 = [v[s] + v[s + 4] for s in range(4)]
            c2 = [b4[s] + b4[s + 2] for s in range(2)]
            pos = (c2[0] + c2[1]).astype(jnp.int32)

            opv = op_v[...]
            prev = jnp.where(pos == 0, SEQ, pos - 1)
            sel = jnp.where(opv != 0, prev, pos)

            def finish(table, init_ref, out_ref):
                idx_v[...] = jnp.maximum(sel - 1, 0) * B + lane
                pltpu.async_copy(table.at[idx_v], rows_v, sem).wait()

                def patch_body(b, carry):
                    bvec = (lane + b) & (B - 1)
                    selb = sel.at[bvec].get(mode="promise_in_bounds")

                    @pl.when(selb[0] == 0)
                    def _():
                        pltpu.sync_copy(init_ref, rows_v.at[b])
                    return carry
                lax.fori_loop(0, B, patch_body, 0)

                pltpu.sync_copy(rows_v, out_ref)

            @pl.when(cid == 0)
            def _():
                finish(h_hbm, ih_hbm, out_h)

            @pl.when(cid == 1)
            def _():
                finish(c_hbm, ic_hbm, out_c)

    return pl.kernel(
        body,
        mesh=mesh,
        out_type=(
            jax.ShapeDtypeStruct((B, H), jnp.float32),
            jax.ShapeDtypeStruct((B, H), jnp.float32),
        ),
        scratch_types=[
            pltpu.VMEM((16, NS * B), jnp.float32),
            pltpu.VMEM((B,), jnp.int32),
            pltpu.VMEM((NS * B,), jnp.float32),
            pltpu.VMEM((NG, NS * B), jnp.float32),
            pltpu.VMEM((B,), jnp.int32),
            pltpu.VMEM((B,), jnp.int32),
            pltpu.VMEM((B, H), jnp.float32),
            pltpu.MemorySpace.VMEM_SHARED((NG, NS * B), jnp.float32),
            pltpu.SemaphoreType.DMA,
        ],
    )


_sc_kernel = _make_kernel()


def kernel(hiddens, cells, hidden_masks, op, init_hidden, init_cell):
    h2 = hiddens.reshape(SEQ * B, H)
    c2 = cells.reshape(SEQ * B, H)
    m2 = hidden_masks.reshape(SEQ * B // 128, 128)
    return _sc_kernel(h2, c2, m2, op, init_hidden, init_cell)
